# Initial kernel scaffold; baseline (speedup 1.0000x reference)
import jax
import jax.numpy as jnp
from jax.experimental import pallas as pl


def kernel(x, edge_index, w1_0, b1_0, w2_0, b2_0, w1_1, b1_1, w2_1, b2_1, w1_2, b1_2, w2_2, b2_2, wc1, bc1, wc2, bc2):
    raise NotImplementedError("write your pallas kernel here")


# speedup vs baseline: 4.1657x; 4.1657x over previous
import functools

import jax
import jax.numpy as jnp
from jax import lax
from jax.experimental import pallas as pl
from jax.experimental.pallas import tpu as pltpu
from jax.experimental.pallas import tpu_sc as plsc

N = 10000
E = 160000
D = 256
DIM1 = 512
R = 3
NCLS = 10
EPS = 0.1

NPAD = 10240
W = 128
NSUB = 16
NCORE = 2
BE = 128
NB = 80
EPT = NB * BE
PE = NSUB * EPT
ROWS_PT = NPAD // NSUB
BN = 512
GRID = NPAD // BN


def _make_segsum(nchunk):
    cpc = nchunk // NCORE
    mesh = plsc.VectorSubcoreMesh(core_axis_name="c", subcore_axis_name="s")

    @functools.partial(
        pl.kernel,
        mesh=mesh,
        out_type=jax.ShapeDtypeStruct((nchunk, NPAD, W), jnp.float32),
        scratch_types=[
            pltpu.VMEM((NB, BE), jnp.int32),
            pltpu.VMEM((NB, BE), jnp.int32),
            pltpu.VMEM((BE, W), jnp.float32),
            pltpu.VMEM_SHARED((NPAD, W), jnp.float32),
            pltpu.SemaphoreType.DMA,
        ],
    )
    def segsum(y4, src4, dst3, zeros, agg, srcv, dstv, rows, acc, gsem):
        c = lax.axis_index("c")
        s = lax.axis_index("s")
        r0 = s * ROWS_PT
        pltpu.sync_copy(dst3.at[s], dstv)
        pltpu.sync_copy(zeros.at[pl.ds(r0, ROWS_PT)], acc.at[pl.ds(r0, ROWS_PT)])
        plsc.subcore_barrier()
        for ci in range(cpc):
            ch = c * cpc + ci
            pltpu.sync_copy(src4.at[ch, s], srcv)

            def batch(b, carry):
                pltpu.async_copy(y4.at[srcv.at[b]], rows, gsem).wait()
                pltpu.sync_copy(rows, acc.at[dstv.at[b]], add=True)
                return carry

            lax.fori_loop(0, NB, batch, 0)
            plsc.subcore_barrier()
            pltpu.sync_copy(acc.at[pl.ds(r0, ROWS_PT)],
                            agg.at[ch, pl.ds(r0, ROWS_PT)])
            if ci + 1 < cpc:
                pltpu.sync_copy(zeros.at[pl.ds(r0, ROWS_PT)],
                                acc.at[pl.ds(r0, ROWS_PT)])
                plsc.subcore_barrier()

    return segsum


_segsum2 = _make_segsum(2)
_segsum4 = _make_segsum(4)


def _mlp_body(nchunk):
    def body(y_ref, agg_ref, w1t_ref, b1_ref, w2t_ref, b2_ref, yn_ref, cs_ref):
        i = pl.program_id(0)
        h1pre = None
        for ci in range(nchunk):
            hin = (1.0 + EPS) * y_ref[:, ci * W:(ci + 1) * W] + agg_ref[ci]
            p = jnp.dot(hin, w1t_ref[ci * W:(ci + 1) * W, :],
                        preferred_element_type=jnp.float32)
            h1pre = p if h1pre is None else h1pre + p
        h1 = jnp.tanh(h1pre + b1_ref[...])
        h2 = jnp.tanh(jnp.dot(h1, w2t_ref[...],
                              preferred_element_type=jnp.float32) + b2_ref[...])
        y = jnp.tanh(h2)
        yn_ref[...] = y
        rows = i * BN + lax.broadcasted_iota(jnp.int32, (BN, 1), 0)
        ym = jnp.where(rows < N, y, 0.0)

        @pl.when(i == 0)
        def _():
            cs_ref[...] = jnp.zeros_like(cs_ref)

        cs_ref[...] += jnp.sum(ym, axis=0, keepdims=True)

    return body


def _make_mlp(nchunk):
    fin = nchunk * W
    return pl.pallas_call(
        _mlp_body(nchunk),
        grid=(GRID,),
        in_specs=[
            pl.BlockSpec((BN, fin), lambda i: (i, 0)),
            pl.BlockSpec((nchunk, BN, W), lambda i: (0, i, 0)),
            pl.BlockSpec((fin, DIM1), lambda i: (0, 0)),
            pl.BlockSpec((1, DIM1), lambda i: (0, 0)),
            pl.BlockSpec((DIM1, DIM1), lambda i: (0, 0)),
            pl.BlockSpec((1, DIM1), lambda i: (0, 0)),
        ],
        out_specs=[
            pl.BlockSpec((BN, DIM1), lambda i: (i, 0)),
            pl.BlockSpec((1, DIM1), lambda i: (0, 0)),
        ],
        out_shape=[
            jax.ShapeDtypeStruct((NPAD, DIM1), jnp.float32),
            jax.ShapeDtypeStruct((1, DIM1), jnp.float32),
        ],
    )


_mlp2 = _make_mlp(2)
_mlp4 = _make_mlp(4)


def _head_body(cols_ref, wc1t_ref, bc1_ref, wc2t_ref, bc2_ref, out_ref):
    acc = None
    for j in range(R):
        p = jnp.dot(cols_ref[j:j + 1, :], wc1t_ref[j * DIM1:(j + 1) * DIM1, :],
                    preferred_element_type=jnp.float32)
        acc = p if acc is None else acc + p
    hidden = jnp.tanh(acc + bc1_ref[...])
    out_ref[...] = jnp.dot(hidden, wc2t_ref[...],
                           preferred_element_type=jnp.float32) + bc2_ref[...]


_head = pl.pallas_call(
    _head_body,
    out_shape=jax.ShapeDtypeStruct((1, NCLS), jnp.float32),
)


def kernel(x, edge_index, w1_0, b1_0, w2_0, b2_0, w1_1, b1_1, w2_1, b2_1,
           w1_2, b1_2, w2_2, b2_2, wc1, bc1, wc2, bc2):
    xp = jnp.pad(x, ((0, NPAD - N), (0, 0)))
    src = edge_index[0]
    dst = edge_index[1]
    pad_ids = N + (jnp.arange(PE - E, dtype=jnp.int32) % 16)
    srcp = jnp.concatenate([src, pad_ids])
    dstp = jnp.concatenate([dst, pad_ids])
    dst3 = dstp.reshape(NSUB, NB, BE)
    idx2 = (srcp[None, :] * 2 +
            jnp.arange(2, dtype=jnp.int32)[:, None]).reshape(2, NSUB, NB, BE)
    idx4 = (srcp[None, :] * 4 +
            jnp.arange(4, dtype=jnp.int32)[:, None]).reshape(4, NSUB, NB, BE)
    zeros = jnp.zeros((NPAD, W), jnp.float32)

    layers = [(w1_0, b1_0, w2_0, b2_0), (w1_1, b1_1, w2_1, b2_1),
              (w1_2, b1_2, w2_2, b2_2)]
    Y = xp
    cols = []
    for j, (w1, b1, w2, b2) in enumerate(layers):
        nchunk = (D if j == 0 else DIM1) // W
        y4 = Y.reshape(NPAD * nchunk, W)
        if nchunk == 2:
            agg = _segsum2(y4, idx2, dst3, zeros)
            Y, cs = _mlp2(Y, agg, w1.T, b1.reshape(1, DIM1), w2.T,
                          b2.reshape(1, DIM1))
        else:
            agg = _segsum4(y4, idx4, dst3, zeros)
            Y, cs = _mlp4(Y, agg, w1.T, b1.reshape(1, DIM1), w2.T,
                          b2.reshape(1, DIM1))
        cols.append(cs)

    colsmat = jnp.concatenate(cols, axis=0)
    wc1p = wc1.reshape(DIM1, DIM1, R).transpose(0, 2, 1).reshape(DIM1, R * DIM1)
    return _head(colsmat, wc1p.T, bc1.reshape(1, DIM1), wc2.T,
                 bc2.reshape(1, NCLS))


# speedup vs baseline: 4.7600x; 1.1427x over previous
import functools

import jax
import jax.numpy as jnp
from jax import lax
from jax.experimental import pallas as pl
from jax.experimental.pallas import tpu as pltpu
from jax.experimental.pallas import tpu_sc as plsc

N = 10000
E = 160000
D = 256
DIM1 = 512
R = 3
NCLS = 10
EPS = 0.1

NPAD = 10240
W = 128
NSUB = 16
NCORE = 2
BE = 128
NB = 80
EPT = NB * BE
PE = NSUB * EPT
ROWS_PT = NPAD // NSUB
BN = 512
GRID = NPAD // BN


def _make_segsum(nchunk):
    cpc = nchunk // NCORE
    mesh = plsc.VectorSubcoreMesh(core_axis_name="c", subcore_axis_name="s")

    @functools.partial(
        pl.kernel,
        mesh=mesh,
        out_type=jax.ShapeDtypeStruct((nchunk, NPAD, W), jnp.float32),
        scratch_types=[
            pltpu.VMEM((NB // 2, BE), jnp.int32),
            pltpu.VMEM((NB // 2, BE), jnp.int32),
            pltpu.VMEM((BE, W), jnp.float32),
            pltpu.VMEM((BE, W), jnp.float32),
            pltpu.VMEM_SHARED((NPAD, W), jnp.float32),
            pltpu.SemaphoreType.DMA,
            pltpu.SemaphoreType.DMA,
            pltpu.SemaphoreType.DMA,
            pltpu.SemaphoreType.DMA,
        ],
    )
    def segsum(y4, src4, dst3, zeros, agg, srcv, dstv, rb0, rb1,
               acc, gs0, gs1, ss0, ss1):
        c = lax.axis_index("c")
        s = lax.axis_index("s")
        r0 = s * ROWS_PT
        pltpu.sync_copy(zeros.at[pl.ds(r0, ROWS_PT)], acc.at[pl.ds(r0, ROWS_PT)])
        plsc.subcore_barrier()
        for ci in range(cpc):
            ch = c * cpc + ci
            for half in range(2):
                h0 = half * (NB // 2)
                pltpu.sync_copy(src4.at[ch, s, pl.ds(h0, NB // 2)], srcv)
                pltpu.sync_copy(dst3.at[s, pl.ds(h0, NB // 2)], dstv)

                def batch2(k, carry):
                    b = k * 2
                    g0 = pltpu.async_copy(y4.at[srcv.at[b]], rb0, gs0)
                    g1 = pltpu.async_copy(y4.at[srcv.at[b + 1]], rb1, gs1)
                    g0.wait()
                    s0 = pltpu.async_copy(rb0, acc.at[dstv.at[b]], ss0,
                                          add=True)
                    g1.wait()
                    s1 = pltpu.async_copy(rb1, acc.at[dstv.at[b + 1]], ss1,
                                          add=True)
                    s0.wait()
                    s1.wait()
                    return carry

                lax.fori_loop(0, NB // 4, batch2, 0)
            plsc.subcore_barrier()
            pltpu.sync_copy(acc.at[pl.ds(r0, ROWS_PT)],
                            agg.at[ch, pl.ds(r0, ROWS_PT)])
            if ci + 1 < cpc:
                pltpu.sync_copy(zeros.at[pl.ds(r0, ROWS_PT)],
                                acc.at[pl.ds(r0, ROWS_PT)])
                plsc.subcore_barrier()

    return segsum


_segsum2 = _make_segsum(2)
_segsum4 = _make_segsum(4)


def _mlp_body(nchunk):
    def body(y_ref, agg_ref, w1t_ref, b1_ref, w2t_ref, b2_ref, yn_ref, cs_ref):
        i = pl.program_id(0)
        h1pre = None
        for ci in range(nchunk):
            hin = (1.0 + EPS) * y_ref[:, ci * W:(ci + 1) * W] + agg_ref[ci]
            p = jnp.dot(hin, w1t_ref[ci * W:(ci + 1) * W, :],
                        preferred_element_type=jnp.float32)
            h1pre = p if h1pre is None else h1pre + p
        h1 = jnp.tanh(h1pre + b1_ref[...])
        h2 = jnp.tanh(jnp.dot(h1, w2t_ref[...],
                              preferred_element_type=jnp.float32) + b2_ref[...])
        y = jnp.tanh(h2)
        yn_ref[...] = y
        rows = i * BN + lax.broadcasted_iota(jnp.int32, (BN, 1), 0)
        ym = jnp.where(rows < N, y, 0.0)

        @pl.when(i == 0)
        def _():
            cs_ref[...] = jnp.zeros_like(cs_ref)

        cs_ref[...] += jnp.sum(ym, axis=0, keepdims=True)

    return body


def _make_mlp(nchunk):
    fin = nchunk * W
    return pl.pallas_call(
        _mlp_body(nchunk),
        grid=(GRID,),
        in_specs=[
            pl.BlockSpec((BN, fin), lambda i: (i, 0)),
            pl.BlockSpec((nchunk, BN, W), lambda i: (0, i, 0)),
            pl.BlockSpec((fin, DIM1), lambda i: (0, 0)),
            pl.BlockSpec((1, DIM1), lambda i: (0, 0)),
            pl.BlockSpec((DIM1, DIM1), lambda i: (0, 0)),
            pl.BlockSpec((1, DIM1), lambda i: (0, 0)),
        ],
        out_specs=[
            pl.BlockSpec((BN, DIM1), lambda i: (i, 0)),
            pl.BlockSpec((1, DIM1), lambda i: (0, 0)),
        ],
        out_shape=[
            jax.ShapeDtypeStruct((NPAD, DIM1), jnp.float32),
            jax.ShapeDtypeStruct((1, DIM1), jnp.float32),
        ],
    )


_mlp2 = _make_mlp(2)
_mlp4 = _make_mlp(4)


def _head_body(cols_ref, wc1t_ref, bc1_ref, wc2t_ref, bc2_ref, out_ref):
    acc = None
    for j in range(R):
        p = jnp.dot(cols_ref[j:j + 1, :], wc1t_ref[j * DIM1:(j + 1) * DIM1, :],
                    preferred_element_type=jnp.float32)
        acc = p if acc is None else acc + p
    hidden = jnp.tanh(acc + bc1_ref[...])
    out_ref[...] = jnp.dot(hidden, wc2t_ref[...],
                           preferred_element_type=jnp.float32) + bc2_ref[...]


_head = pl.pallas_call(
    _head_body,
    out_shape=jax.ShapeDtypeStruct((1, NCLS), jnp.float32),
)


def kernel(x, edge_index, w1_0, b1_0, w2_0, b2_0, w1_1, b1_1, w2_1, b2_1,
           w1_2, b1_2, w2_2, b2_2, wc1, bc1, wc2, bc2):
    xp = jnp.pad(x, ((0, NPAD - N), (0, 0)))
    src = edge_index[0]
    dst = edge_index[1]
    pad_ids = N + (jnp.arange(PE - E, dtype=jnp.int32) % 16)
    srcp = jnp.concatenate([src, pad_ids])
    dstp = jnp.concatenate([dst, pad_ids])
    dst3 = dstp.reshape(NSUB, NB, BE)
    idx2 = (srcp[None, :] * 2 +
            jnp.arange(2, dtype=jnp.int32)[:, None]).reshape(2, NSUB, NB, BE)
    idx4 = (srcp[None, :] * 4 +
            jnp.arange(4, dtype=jnp.int32)[:, None]).reshape(4, NSUB, NB, BE)
    zeros = jnp.zeros((NPAD, W), jnp.float32)

    layers = [(w1_0, b1_0, w2_0, b2_0), (w1_1, b1_1, w2_1, b2_1),
              (w1_2, b1_2, w2_2, b2_2)]
    Y = xp
    cols = []
    for j, (w1, b1, w2, b2) in enumerate(layers):
        nchunk = (D if j == 0 else DIM1) // W
        y4 = Y.reshape(NPAD * nchunk, W)
        if nchunk == 2:
            agg = _segsum2(y4, idx2, dst3, zeros)
            Y, cs = _mlp2(Y, agg, w1.T, b1.reshape(1, DIM1), w2.T,
                          b2.reshape(1, DIM1))
        else:
            agg = _segsum4(y4, idx4, dst3, zeros)
            Y, cs = _mlp4(Y, agg, w1.T, b1.reshape(1, DIM1), w2.T,
                          b2.reshape(1, DIM1))
        cols.append(cs)

    colsmat = jnp.concatenate(cols, axis=0)
    wc1p = wc1.reshape(DIM1, DIM1, R).transpose(0, 2, 1).reshape(DIM1, R * DIM1)
    return _head(colsmat, wc1p.T, bc1.reshape(1, DIM1), wc2.T,
                 bc2.reshape(1, NCLS))


# speedup vs baseline: 4.7769x; 1.0035x over previous
import functools

import jax
import jax.numpy as jnp
from jax import lax
from jax.experimental import pallas as pl
from jax.experimental.pallas import tpu as pltpu
from jax.experimental.pallas import tpu_sc as plsc

N = 10000
E = 160000
D = 256
DIM1 = 512
R = 3
NCLS = 10
EPS = 0.1

NPAD = 10240
W = 128
NSUB = 16
NCORE = 2
BE = 128
NB = 80
EPT = NB * BE
PE = NSUB * EPT
ROWS_PT = NPAD // NSUB
BN = 512
GRID = NPAD // BN


def _make_segsum(nchunk):
    cpc = nchunk // NCORE
    mesh = plsc.VectorSubcoreMesh(core_axis_name="c", subcore_axis_name="s")

    @functools.partial(
        pl.kernel,
        mesh=mesh,
        out_type=jax.ShapeDtypeStruct((nchunk, NPAD, W), jnp.float32),
        scratch_types=[
            pltpu.VMEM((NB // 2, BE), jnp.int32),
            pltpu.VMEM((NB // 2, BE), jnp.int32),
            pltpu.VMEM((BE, W), jnp.float32),
            pltpu.VMEM((BE, W), jnp.float32),
            pltpu.VMEM_SHARED((NPAD, W), jnp.float32),
            pltpu.SemaphoreType.DMA,
            pltpu.SemaphoreType.DMA,
            pltpu.SemaphoreType.DMA,
            pltpu.SemaphoreType.DMA,
        ],
    )
    def segsum(y4, src4, dst3, zeros, agg, srcv, dstv, rb0, rb1,
               acc, gs0, gs1, ss0, ss1):
        c = lax.axis_index("c")
        s = lax.axis_index("s")
        r0 = s * ROWS_PT
        pltpu.sync_copy(zeros.at[pl.ds(r0, ROWS_PT)], acc.at[pl.ds(r0, ROWS_PT)])
        plsc.subcore_barrier()
        for ci in range(cpc):
            ch = c * cpc + ci
            for half in range(2):
                h0 = half * (NB // 2)
                NBH = NB // 2
                pltpu.sync_copy(src4.at[ch, s, pl.ds(h0, NBH)], srcv)
                pltpu.sync_copy(dst3.at[s, pl.ds(h0, NBH)], dstv)
                pltpu.async_copy(y4.at[srcv.at[0]], rb0, gs0)
                pltpu.async_copy(y4.at[srcv.at[1]], rb1, gs1)

                def batch2(k, carry):
                    b = 2 * k
                    nxt0 = jnp.where(b + 2 < NBH, b + 2, 0)
                    nxt1 = jnp.where(b + 3 < NBH, b + 3, 1)
                    pltpu.make_async_copy(y4.at[pl.ds(0, BE)], rb0, gs0).wait()
                    pltpu.async_copy(rb0, acc.at[dstv.at[b]], ss0, add=True)
                    pltpu.make_async_copy(y4.at[pl.ds(0, BE)], rb1, gs1).wait()
                    pltpu.async_copy(rb1, acc.at[dstv.at[b + 1]], ss1,
                                     add=True)
                    pltpu.make_async_copy(rb0, acc.at[pl.ds(0, BE)],
                                          ss0).wait()
                    pltpu.async_copy(y4.at[srcv.at[nxt0]], rb0, gs0)
                    pltpu.make_async_copy(rb1, acc.at[pl.ds(0, BE)],
                                          ss1).wait()
                    pltpu.async_copy(y4.at[srcv.at[nxt1]], rb1, gs1)
                    return carry

                lax.fori_loop(0, NBH // 2, batch2, 0)
                pltpu.make_async_copy(y4.at[pl.ds(0, BE)], rb0, gs0).wait()
                pltpu.make_async_copy(y4.at[pl.ds(0, BE)], rb1, gs1).wait()
            plsc.subcore_barrier()
            pltpu.sync_copy(acc.at[pl.ds(r0, ROWS_PT)],
                            agg.at[ch, pl.ds(r0, ROWS_PT)])
            if ci + 1 < cpc:
                pltpu.sync_copy(zeros.at[pl.ds(r0, ROWS_PT)],
                                acc.at[pl.ds(r0, ROWS_PT)])
                plsc.subcore_barrier()

    return segsum


_segsum2 = _make_segsum(2)
_segsum4 = _make_segsum(4)


def _mlp_body(nchunk):
    def body(y_ref, agg_ref, w1t_ref, b1_ref, w2t_ref, b2_ref, yn_ref, cs_ref):
        i = pl.program_id(0)
        h1pre = None
        for ci in range(nchunk):
            hin = (1.0 + EPS) * y_ref[:, ci * W:(ci + 1) * W] + agg_ref[ci]
            p = jnp.dot(hin, w1t_ref[ci * W:(ci + 1) * W, :],
                        preferred_element_type=jnp.float32)
            h1pre = p if h1pre is None else h1pre + p
        h1 = jnp.tanh(h1pre + b1_ref[...])
        h2 = jnp.tanh(jnp.dot(h1, w2t_ref[...],
                              preferred_element_type=jnp.float32) + b2_ref[...])
        y = jnp.tanh(h2)
        yn_ref[...] = y
        rows = i * BN + lax.broadcasted_iota(jnp.int32, (BN, 1), 0)
        ym = jnp.where(rows < N, y, 0.0)

        @pl.when(i == 0)
        def _():
            cs_ref[...] = jnp.zeros_like(cs_ref)

        cs_ref[...] += jnp.sum(ym, axis=0, keepdims=True)

    return body


def _make_mlp(nchunk):
    fin = nchunk * W
    return pl.pallas_call(
        _mlp_body(nchunk),
        grid=(GRID,),
        in_specs=[
            pl.BlockSpec((BN, fin), lambda i: (i, 0)),
            pl.BlockSpec((nchunk, BN, W), lambda i: (0, i, 0)),
            pl.BlockSpec((fin, DIM1), lambda i: (0, 0)),
            pl.BlockSpec((1, DIM1), lambda i: (0, 0)),
            pl.BlockSpec((DIM1, DIM1), lambda i: (0, 0)),
            pl.BlockSpec((1, DIM1), lambda i: (0, 0)),
        ],
        out_specs=[
            pl.BlockSpec((BN, DIM1), lambda i: (i, 0)),
            pl.BlockSpec((1, DIM1), lambda i: (0, 0)),
        ],
        out_shape=[
            jax.ShapeDtypeStruct((NPAD, DIM1), jnp.float32),
            jax.ShapeDtypeStruct((1, DIM1), jnp.float32),
        ],
    )


_mlp2 = _make_mlp(2)
_mlp4 = _make_mlp(4)


def _head_body(cols_ref, wc1t_ref, bc1_ref, wc2t_ref, bc2_ref, out_ref):
    acc = None
    for j in range(R):
        p = jnp.dot(cols_ref[j:j + 1, :], wc1t_ref[j * DIM1:(j + 1) * DIM1, :],
                    preferred_element_type=jnp.float32)
        acc = p if acc is None else acc + p
    hidden = jnp.tanh(acc + bc1_ref[...])
    out_ref[...] = jnp.dot(hidden, wc2t_ref[...],
                           preferred_element_type=jnp.float32) + bc2_ref[...]


_head = pl.pallas_call(
    _head_body,
    out_shape=jax.ShapeDtypeStruct((1, NCLS), jnp.float32),
)


def kernel(x, edge_index, w1_0, b1_0, w2_0, b2_0, w1_1, b1_1, w2_1, b2_1,
           w1_2, b1_2, w2_2, b2_2, wc1, bc1, wc2, bc2):
    xp = jnp.pad(x, ((0, NPAD - N), (0, 0)))
    src = edge_index[0]
    dst = edge_index[1]
    pad_ids = N + (jnp.arange(PE - E, dtype=jnp.int32) % 16)
    srcp = jnp.concatenate([src, pad_ids])
    dstp = jnp.concatenate([dst, pad_ids])
    dst3 = dstp.reshape(NSUB, NB, BE)
    idx2 = (srcp[None, :] * 2 +
            jnp.arange(2, dtype=jnp.int32)[:, None]).reshape(2, NSUB, NB, BE)
    idx4 = (srcp[None, :] * 4 +
            jnp.arange(4, dtype=jnp.int32)[:, None]).reshape(4, NSUB, NB, BE)
    zeros = jnp.zeros((NPAD, W), jnp.float32)

    layers = [(w1_0, b1_0, w2_0, b2_0), (w1_1, b1_1, w2_1, b2_1),
              (w1_2, b1_2, w2_2, b2_2)]
    Y = xp
    cols = []
    for j, (w1, b1, w2, b2) in enumerate(layers):
        nchunk = (D if j == 0 else DIM1) // W
        y4 = Y.reshape(NPAD * nchunk, W)
        if nchunk == 2:
            agg = _segsum2(y4, idx2, dst3, zeros)
            Y, cs = _mlp2(Y, agg, w1.T, b1.reshape(1, DIM1), w2.T,
                          b2.reshape(1, DIM1))
        else:
            agg = _segsum4(y4, idx4, dst3, zeros)
            Y, cs = _mlp4(Y, agg, w1.T, b1.reshape(1, DIM1), w2.T,
                          b2.reshape(1, DIM1))
        cols.append(cs)

    colsmat = jnp.concatenate(cols, axis=0)
    wc1p = wc1.reshape(DIM1, DIM1, R).transpose(0, 2, 1).reshape(DIM1, R * DIM1)
    return _head(colsmat, wc1p.T, bc1.reshape(1, DIM1), wc2.T,
                 bc2.reshape(1, NCLS))


# speedup vs baseline: 4.8664x; 1.0187x over previous
import functools

import jax
import jax.numpy as jnp
from jax import lax
from jax.experimental import pallas as pl
from jax.experimental.pallas import tpu as pltpu
from jax.experimental.pallas import tpu_sc as plsc

N = 10000
E = 160000
D = 256
DIM1 = 512
R = 3
NCLS = 10
EPS = 0.1

NPAD = 10240
W = 128
NSUB = 16
NCORE = 2
BE = 128
NB = 80
EPT = NB * BE
PE = NSUB * EPT
ROWS_PT = NPAD // NSUB
BN = 1024
GRID = NPAD // BN


def _make_segsum(nchunk):
    cpc = nchunk // NCORE
    mesh = plsc.VectorSubcoreMesh(core_axis_name="c", subcore_axis_name="s")

    @functools.partial(
        pl.kernel,
        mesh=mesh,
        out_type=jax.ShapeDtypeStruct((nchunk, NPAD, W), jnp.float32),
        scratch_types=[
            pltpu.VMEM((NB // 2, BE), jnp.int32),
            pltpu.VMEM((NB // 2, BE), jnp.int32),
            pltpu.VMEM((BE, W), jnp.float32),
            pltpu.VMEM((BE, W), jnp.float32),
            pltpu.VMEM_SHARED((NPAD, W), jnp.float32),
            pltpu.SemaphoreType.DMA,
            pltpu.SemaphoreType.DMA,
            pltpu.SemaphoreType.DMA,
            pltpu.SemaphoreType.DMA,
        ],
    )
    def segsum(y4, src4, dst3, zeros, agg, srcv, dstv, rb0, rb1,
               acc, gs0, gs1, ss0, ss1):
        c = lax.axis_index("c")
        s = lax.axis_index("s")
        r0 = s * ROWS_PT
        pltpu.sync_copy(zeros.at[pl.ds(r0, ROWS_PT)], acc.at[pl.ds(r0, ROWS_PT)])
        plsc.subcore_barrier()
        for ci in range(cpc):
            ch = c * cpc + ci
            for half in range(2):
                h0 = half * (NB // 2)
                NBH = NB // 2
                pltpu.sync_copy(src4.at[ch, s, pl.ds(h0, NBH)], srcv)
                pltpu.sync_copy(dst3.at[s, pl.ds(h0, NBH)], dstv)
                pltpu.async_copy(y4.at[srcv.at[0]], rb0, gs0)
                pltpu.async_copy(y4.at[srcv.at[1]], rb1, gs1)

                def batch2(k, carry):
                    b = 2 * k
                    nxt0 = jnp.where(b + 2 < NBH, b + 2, 0)
                    nxt1 = jnp.where(b + 3 < NBH, b + 3, 1)
                    pltpu.make_async_copy(y4.at[pl.ds(0, BE)], rb0, gs0).wait()
                    pltpu.async_copy(rb0, acc.at[dstv.at[b]], ss0, add=True)
                    pltpu.make_async_copy(y4.at[pl.ds(0, BE)], rb1, gs1).wait()
                    pltpu.async_copy(rb1, acc.at[dstv.at[b + 1]], ss1,
                                     add=True)
                    pltpu.make_async_copy(rb0, acc.at[pl.ds(0, BE)],
                                          ss0).wait()
                    pltpu.async_copy(y4.at[srcv.at[nxt0]], rb0, gs0)
                    pltpu.make_async_copy(rb1, acc.at[pl.ds(0, BE)],
                                          ss1).wait()
                    pltpu.async_copy(y4.at[srcv.at[nxt1]], rb1, gs1)
                    return carry

                lax.fori_loop(0, NBH // 2, batch2, 0)
                pltpu.make_async_copy(y4.at[pl.ds(0, BE)], rb0, gs0).wait()
                pltpu.make_async_copy(y4.at[pl.ds(0, BE)], rb1, gs1).wait()
            plsc.subcore_barrier()
            pltpu.sync_copy(acc.at[pl.ds(r0, ROWS_PT)],
                            agg.at[ch, pl.ds(r0, ROWS_PT)])
            if ci + 1 < cpc:
                pltpu.sync_copy(zeros.at[pl.ds(r0, ROWS_PT)],
                                acc.at[pl.ds(r0, ROWS_PT)])
                plsc.subcore_barrier()

    return segsum


_segsum2 = _make_segsum(2)
_segsum4 = _make_segsum(4)


def _mlp_body(nchunk):
    def body(y_ref, agg_ref, w1t_ref, b1_ref, w2t_ref, b2_ref, yn_ref, cs_ref):
        i = pl.program_id(0)
        h1pre = None
        for ci in range(nchunk):
            hin = (1.0 + EPS) * y_ref[:, ci * W:(ci + 1) * W] + agg_ref[ci]
            p = jnp.dot(hin, w1t_ref[ci * W:(ci + 1) * W, :],
                        preferred_element_type=jnp.float32)
            h1pre = p if h1pre is None else h1pre + p
        h1 = jnp.tanh(h1pre + b1_ref[...])
        h2 = jnp.tanh(jnp.dot(h1, w2t_ref[...],
                              preferred_element_type=jnp.float32) + b2_ref[...])
        y = jnp.tanh(h2)
        yn_ref[...] = y
        rows = i * BN + lax.broadcasted_iota(jnp.int32, (BN, 1), 0)
        ym = jnp.where(rows < N, y, 0.0)

        @pl.when(i == 0)
        def _():
            cs_ref[...] = jnp.zeros_like(cs_ref)

        cs_ref[...] += jnp.sum(ym, axis=0, keepdims=True)

    return body


def _make_mlp(nchunk):
    fin = nchunk * W
    return pl.pallas_call(
        _mlp_body(nchunk),
        grid=(GRID,),
        in_specs=[
            pl.BlockSpec((BN, fin), lambda i: (i, 0)),
            pl.BlockSpec((nchunk, BN, W), lambda i: (0, i, 0)),
            pl.BlockSpec((fin, DIM1), lambda i: (0, 0)),
            pl.BlockSpec((1, DIM1), lambda i: (0, 0)),
            pl.BlockSpec((DIM1, DIM1), lambda i: (0, 0)),
            pl.BlockSpec((1, DIM1), lambda i: (0, 0)),
        ],
        out_specs=[
            pl.BlockSpec((BN, DIM1), lambda i: (i, 0)),
            pl.BlockSpec((1, DIM1), lambda i: (0, 0)),
        ],
        out_shape=[
            jax.ShapeDtypeStruct((NPAD, DIM1), jnp.float32),
            jax.ShapeDtypeStruct((1, DIM1), jnp.float32),
        ],
    )


_mlp2 = _make_mlp(2)
_mlp4 = _make_mlp(4)


def _head_body(cols_ref, wc1t_ref, bc1_ref, wc2t_ref, bc2_ref, out_ref):
    acc = None
    for j in range(R):
        p = jnp.dot(cols_ref[j:j + 1, :], wc1t_ref[j * DIM1:(j + 1) * DIM1, :],
                    preferred_element_type=jnp.float32)
        acc = p if acc is None else acc + p
    hidden = jnp.tanh(acc + bc1_ref[...])
    out_ref[...] = jnp.dot(hidden, wc2t_ref[...],
                           preferred_element_type=jnp.float32) + bc2_ref[...]


_head = pl.pallas_call(
    _head_body,
    out_shape=jax.ShapeDtypeStruct((1, NCLS), jnp.float32),
)


def kernel(x, edge_index, w1_0, b1_0, w2_0, b2_0, w1_1, b1_1, w2_1, b2_1,
           w1_2, b1_2, w2_2, b2_2, wc1, bc1, wc2, bc2):
    xp = jnp.pad(x, ((0, NPAD - N), (0, 0)))
    src = edge_index[0]
    dst = edge_index[1]
    pad_ids = N + (jnp.arange(PE - E, dtype=jnp.int32) % 16)
    srcp = jnp.concatenate([src, pad_ids])
    dstp = jnp.concatenate([dst, pad_ids])
    dst3 = dstp.reshape(NSUB, NB, BE)
    idx2 = (srcp[None, :] * 2 +
            jnp.arange(2, dtype=jnp.int32)[:, None]).reshape(2, NSUB, NB, BE)
    idx4 = (srcp[None, :] * 4 +
            jnp.arange(4, dtype=jnp.int32)[:, None]).reshape(4, NSUB, NB, BE)
    zeros = jnp.zeros((NPAD, W), jnp.float32)

    layers = [(w1_0, b1_0, w2_0, b2_0), (w1_1, b1_1, w2_1, b2_1),
              (w1_2, b1_2, w2_2, b2_2)]
    Y = xp
    cols = []
    for j, (w1, b1, w2, b2) in enumerate(layers):
        nchunk = (D if j == 0 else DIM1) // W
        y4 = Y.reshape(NPAD * nchunk, W)
        if nchunk == 2:
            agg = _segsum2(y4, idx2, dst3, zeros)
            Y, cs = _mlp2(Y, agg, w1.T, b1.reshape(1, DIM1), w2.T,
                          b2.reshape(1, DIM1))
        else:
            agg = _segsum4(y4, idx4, dst3, zeros)
            Y, cs = _mlp4(Y, agg, w1.T, b1.reshape(1, DIM1), w2.T,
                          b2.reshape(1, DIM1))
        cols.append(cs)

    colsmat = jnp.concatenate(cols, axis=0)
    wc1p = wc1.reshape(DIM1, DIM1, R).transpose(0, 2, 1).reshape(DIM1, R * DIM1)
    return _head(colsmat, wc1p.T, bc1.reshape(1, DIM1), wc2.T,
                 bc2.reshape(1, NCLS))
